# baseline (speedup 1.0000x reference)
import functools

import jax
import jax.numpy as jnp
from jax import lax
from jax.experimental import pallas as pl
from jax.experimental.pallas import tpu as pltpu
from jax.experimental.pallas import tpu_sc as plsc

VOCAB = 1000000
DIM = 64
BATCH = 16384

_info = plsc.get_sparse_core_info()
_NC, _NS, _L = _info.num_cores, _info.num_subcores, _info.num_lanes
_NW = _NC * _NS
_BPW = BATCH // _NW
_CHUNK = 128
_NCH = _BPW // _CHUNK
_GPC = _CHUNK // _L

_VBLK = 1024
_GRID = (VOCAB + _VBLK - 1) // _VBLK
_PROWS = _GRID * (_VBLK // 2)


def _tc_pack_body(wt_ref, wc_ref, pt_ref, pc_ref):
    row = lax.broadcasted_iota(jnp.int32, (DIM, DIM), 0)
    col = lax.broadcasted_iota(jnp.int32, (DIM, DIM), 1)
    ident = (row == col).astype(jnp.float32)
    for src, dst in ((wt_ref, pt_ref), (wc_ref, pc_ref)):
        x = src[...]
        xt = lax.dot_general(
            x, ident, (((0,), (0,)), ((), ())),
            preferred_element_type=jnp.float32)
        dst[:, 0:DIM] = xt[0:_VBLK // 2]
        dst[:, DIM:2 * DIM] = xt[_VBLK // 2:_VBLK]


def _pack_tables(wtT, wcT):
    return pl.pallas_call(
        _tc_pack_body,
        grid=(_GRID,),
        in_specs=[
            pl.BlockSpec((DIM, _VBLK), lambda i: (0, i)),
            pl.BlockSpec((DIM, _VBLK), lambda i: (0, i)),
        ],
        out_specs=[
            pl.BlockSpec((_VBLK // 2, 2 * DIM), lambda i: (i, 0)),
            pl.BlockSpec((_VBLK // 2, 2 * DIM), lambda i: (i, 0)),
        ],
        out_shape=[
            jax.ShapeDtypeStruct((_PROWS, 2 * DIM), jnp.float32),
            jax.ShapeDtypeStruct((_PROWS, 2 * DIM), jnp.float32),
        ],
    )(wtT, wcT)


def _sc_body(pit_hbm, pic_hbm, hot_hbm, hoc_hbm, pt_hbm, pc_hbm, out_hbm,
             pit_v, pic_v, hot_v, hoc_v, te0, te1, ce0, ce1, dots_v,
             semt, semc):
    wid = lax.axis_index("s") * _NC + lax.axis_index("c")
    base = wid * _BPW

    pltpu.sync_copy(pit_hbm.at[wid], pit_v)
    pltpu.sync_copy(pic_hbm.at[wid], pic_v)
    pltpu.sync_copy(hot_hbm.at[wid], hot_v)
    pltpu.sync_copy(hoc_hbm.at[wid], hoc_v)

    te_b = (te0, te1)
    ce_b = (ce0, ce1)
    lane = lax.iota(jnp.int32, _L)

    def fire(p):
        ht = pltpu.async_copy(pt_hbm.at[pit_v.at[p]], te_b[p % 2], semt)
        hc = pltpu.async_copy(pc_hbm.at[pic_v.at[p]], ce_b[p % 2], semc)
        return ht, hc

    def compute(p):
        te, ce = te_b[p % 2], ce_b[p % 2]

        def group_body(g, carry):
            gbase = p * _CHUNK + g * _L
            rows = g * _L + lane
            ht = hot_v[pl.ds(gbase, _L)]
            hc = hoc_v[pl.ds(gbase, _L)]
            acc = jnp.zeros((_L,), jnp.float32)
            for j in range(DIM):
                tv = plsc.load_gather(te, [rows, ht + j])
                cv = plsc.load_gather(ce, [rows, hc + j])
                acc = acc + tv * cv
            dots_v[pl.ds(gbase, _L)] = acc
            return carry

        lax.fori_loop(0, _GPC, group_body, 0)

    pending = fire(0)
    for p in range(_NCH):
        nxt = fire(p + 1) if p + 1 < _NCH else None
        pending[0].wait()
        pending[1].wait()
        compute(p)
        pending = nxt

    pltpu.sync_copy(dots_v, out_hbm.at[pl.ds(base, _BPW)])


@jax.jit
def kernel(target, context, W_t, W_c):
    pt, pc = _pack_tables(W_t.T, W_c.T)

    def prep(idx):
        v = idx.reshape(-1).astype(jnp.int32)
        p = (v >> 10) * (_VBLK // 2) + (v & (_VBLK // 2 - 1))
        hoff = ((v >> 9) & 1) * DIM
        return p.reshape(_NW, _NCH, _CHUNK), hoff.reshape(_NW, _BPW)

    pit, hot = prep(target)
    pic, hoc = prep(context)

    run = functools.partial(
        pl.kernel,
        out_type=jax.ShapeDtypeStruct((BATCH,), jnp.float32),
        mesh=plsc.VectorSubcoreMesh(core_axis_name="c", subcore_axis_name="s"),
        compiler_params=pltpu.CompilerParams(
            needs_layout_passes=False, use_tc_tiling_on_sc=True),
        scratch_types=[
            pltpu.VMEM((_NCH, _CHUNK), jnp.int32),
            pltpu.VMEM((_NCH, _CHUNK), jnp.int32),
            pltpu.VMEM((_BPW,), jnp.int32),
            pltpu.VMEM((_BPW,), jnp.int32),
            pltpu.VMEM((_CHUNK, 2 * DIM), jnp.float32),
            pltpu.VMEM((_CHUNK, 2 * DIM), jnp.float32),
            pltpu.VMEM((_CHUNK, 2 * DIM), jnp.float32),
            pltpu.VMEM((_CHUNK, 2 * DIM), jnp.float32),
            pltpu.VMEM((_BPW,), jnp.float32),
            pltpu.SemaphoreType.DMA,
            pltpu.SemaphoreType.DMA,
        ],
    )(_sc_body)
    dots = run(pit, pic, hot, hoc, pt, pc)
    return dots.reshape(BATCH, 1)


# speedup vs baseline: 2.4355x; 2.4355x over previous
import functools

import jax
import jax.numpy as jnp
from jax import lax
from jax.experimental import pallas as pl
from jax.experimental.pallas import tpu as pltpu
from jax.experimental.pallas import tpu_sc as plsc

VOCAB = 1000000
DIM = 64
BATCH = 16384

_info = plsc.get_sparse_core_info()
_NC, _NS, _L = _info.num_cores, _info.num_subcores, _info.num_lanes
_NW = _NC * _NS
_BPW = BATCH // _NW
_RING = 4
_TILE = 128


def _sc_body(vt_hbm, vc_hbm, wt_hbm, wc_hbm, out_hbm,
             vt_v, vc_v, dots_v,
             t0, t1, t2, t3, c0, c1, c2, c3,
             st0, st1, st2, st3, sc0, sc1, sc2, sc3):
    wid = lax.axis_index("s") * _NC + lax.axis_index("c")
    base = wid * _BPW

    pltpu.sync_copy(vt_hbm.at[wid], vt_v)
    pltpu.sync_copy(vc_hbm.at[wid], vc_v)

    t_bufs = (t0, t1, t2, t3)
    c_bufs = (c0, c1, c2, c3)
    t_sems = (st0, st1, st2, st3)
    c_sems = (sc0, sc1, sc2, sc3)
    lane = lax.iota(jnp.int32, _L)

    def scalar_at(ref, i):
        chunk_base = (i >> 4) << 4
        chunk = ref[pl.ds(chunk_base, _L)]
        sel = jnp.where(lane == (i - chunk_base), chunk, 0)
        return jnp.sum(sel)

    def fire(tab, vref, row, buf, sem):
        v = scalar_at(vref, jnp.minimum(row, _BPW - 1))
        off = pl.multiple_of((v >> 7) << 7, _TILE)
        pltpu.async_copy(tab.at[:, pl.ds(off, _TILE)], buf, sem)
        return v & (_TILE - 1)

    def drain(tab, buf, sem):
        pltpu.make_async_copy(tab.at[:, pl.ds(0, _TILE)], buf, sem).wait()

    cols = []
    for s in range(_RING):
        ct = fire(wt_hbm, vt_v, s, t_bufs[s], t_sems[s])
        cc = fire(wc_hbm, vc_v, s, c_bufs[s], c_sems[s])
        cols.extend((ct, cc))

    def body(k, carry):
        *colc, accv = carry
        colc = list(colc)
        for s in range(_RING):
            row = k * _RING + s
            drain(wt_hbm, t_bufs[s], t_sems[s])
            drain(wc_hbm, c_bufs[s], c_sems[s])
            ct = jnp.full((_L,), 0, jnp.int32) + colc[2 * s]
            cc = jnp.full((_L,), 0, jnp.int32) + colc[2 * s + 1]
            acc = jnp.zeros((_L,), jnp.float32)
            for kk in range(DIM // _L):
                rows16 = lane + kk * _L
                tv = plsc.load_gather(t_bufs[s], [rows16, ct])
                cv = plsc.load_gather(c_bufs[s], [rows16, cc])
                acc = acc + tv * cv
            accv = jnp.where(lane == (row & (_L - 1)), jnp.sum(acc), accv)
            colc[2 * s] = fire(wt_hbm, vt_v, row + _RING,
                               t_bufs[s], t_sems[s])
            colc[2 * s + 1] = fire(wc_hbm, vc_v, row + _RING,
                                   c_bufs[s], c_sems[s])
        last = k * _RING + _RING - 1
        dots_v[pl.ds((last >> 4) << 4, _L)] = accv
        return tuple(colc) + (accv,)

    lax.fori_loop(0, _BPW // _RING, body,
                  tuple(cols) + (jnp.zeros((_L,), jnp.float32),))

    for s in range(_RING):
        drain(wt_hbm, t_bufs[s], t_sems[s])
        drain(wc_hbm, c_bufs[s], c_sems[s])

    pltpu.sync_copy(dots_v, out_hbm.at[pl.ds(base, _BPW)])


@jax.jit
def kernel(target, context, W_t, W_c):
    vt = target.reshape(_NW, _BPW).astype(jnp.int32)
    vc = context.reshape(_NW, _BPW).astype(jnp.int32)

    run = functools.partial(
        pl.kernel,
        out_type=jax.ShapeDtypeStruct((BATCH,), jnp.float32),
        mesh=plsc.VectorSubcoreMesh(core_axis_name="c", subcore_axis_name="s"),
        compiler_params=pltpu.CompilerParams(
            needs_layout_passes=False, use_tc_tiling_on_sc=True),
        scratch_types=[
            pltpu.VMEM((_BPW,), jnp.int32),
            pltpu.VMEM((_BPW,), jnp.int32),
            pltpu.VMEM((_BPW,), jnp.float32),
        ] + [pltpu.VMEM((DIM, _TILE), jnp.float32)] * (2 * _RING)
          + [pltpu.SemaphoreType.DMA] * (2 * _RING),
    )(_sc_body)
    dots = run(vt, vc, W_t.T, W_c.T)
    return dots.reshape(BATCH, 1)
